# baseline (speedup 1.0000x reference)
import jax
import jax.numpy as jnp
from jax import lax
from jax.experimental import pallas as pl
from jax.experimental.pallas import tpu as pltpu
from jax.experimental.pallas import tpu_sc as plsc

NUM_ROWS = 100000
EMB_DIM = 3072
BATCH = 8192
CHANNEL, IM_H, IM_W = 3, 32, 32

NUM_CORES = 2
NUM_SUBCORES = 16
NUM_WORKERS = NUM_CORES * NUM_SUBCORES
ROWS_PER_WORKER = BATCH // NUM_WORKERS
BBLK = 128
FBLK = 128
NVB = ROWS_PER_WORKER // BBLK
NJ = EMB_DIM // FBLK


def _sc_body(idx_hbm, table_hbm, out_hbm, idx_v, g0, g1, t0, t1,
             gs0, gs1, ws0, ws1):
    wid = lax.axis_index("s") * NUM_CORES + lax.axis_index("c")
    base = wid * ROWS_PER_WORKER
    pltpu.sync_copy(idx_hbm.at[pl.ds(base, ROWS_PER_WORKER)], idx_v)

    lanes = lax.iota(jnp.int32, 16)
    gbufs, gsems = (g0, g1), (gs0, gs1)
    tbufs, wsems = (t0, t1), (ws0, ws1)

    def gather_desc(v, j, p):
        return pltpu.make_async_copy(
            table_hbm.at[idx_v.at[pl.ds(v * BBLK, BBLK)],
                         pl.ds(j * FBLK, FBLK)],
            gbufs[p], gsems[p])

    def write_desc(v, j, p):
        return pltpu.make_async_copy(
            tbufs[p],
            out_hbm.at[pl.ds(j * FBLK, FBLK),
                       pl.ds(base + v * BBLK, BBLK)],
            wsems[p])

    def transpose(g, t):
        @pl.loop(0, FBLK, unroll=4)
        def _(f):
            col = jnp.full((16,), f, jnp.int32)
            for bg in range(8):
                vv = plsc.load_gather(g, [bg * 16 + lanes, col])
                t[f, pl.ds(bg * 16, 16)] = vv

    for v in range(NVB):
        gather_desc(v, 0, 0).start()
        gather_desc(v, 1, 1).start()

        @pl.loop(0, NJ, step=2)
        def _(j0, v=v):
            for d in range(2):
                j, p = j0 + d, d
                gather_desc(v, j, p).wait()

                @pl.when(j0 >= 2)
                def _():
                    write_desc(v, j, p).wait()

                transpose(gbufs[p], tbufs[p])
                write_desc(v, j, p).start()

                @pl.when(j + 2 < NJ)
                def _():
                    gather_desc(v, j + 2, p).start()

        for p in range(2):
            write_desc(v, 0, p).wait()


def kernel(indices, table):
    idx = indices.astype(jnp.int32)
    mesh = plsc.VectorSubcoreMesh(core_axis_name="c", subcore_axis_name="s")
    out_t = pl.kernel(
        _sc_body,
        out_type=jax.ShapeDtypeStruct((EMB_DIM, BATCH), jnp.float32),
        mesh=mesh,
        scratch_types=[
            pltpu.VMEM((ROWS_PER_WORKER,), jnp.int32),
            pltpu.VMEM((BBLK, FBLK), jnp.float32),
            pltpu.VMEM((BBLK, FBLK), jnp.float32),
            pltpu.VMEM((FBLK, BBLK), jnp.float32),
            pltpu.VMEM((FBLK, BBLK), jnp.float32),
            pltpu.SemaphoreType.DMA,
            pltpu.SemaphoreType.DMA,
            pltpu.SemaphoreType.DMA,
            pltpu.SemaphoreType.DMA,
        ],
        compiler_params=pltpu.CompilerParams(needs_layout_passes=False),
    )(idx, table)
    return out_t.T.reshape(BATCH, CHANNEL, IM_H, IM_W)


# speedup vs baseline: 1.9802x; 1.9802x over previous
import jax
import jax.numpy as jnp
from jax import lax
from jax.experimental import pallas as pl
from jax.experimental.pallas import tpu as pltpu
from jax.experimental.pallas import tpu_sc as plsc

NUM_ROWS = 100000
EMB_DIM = 3072
BATCH = 8192
CHANNEL, IM_H, IM_W = 3, 32, 32

NUM_CORES = 2
NUM_SUBCORES = 16
NUM_WORKERS = NUM_CORES * NUM_SUBCORES
ROWS_PER_WORKER = BATCH // NUM_WORKERS

SPLITS = (512, 1024, 1024, 512)

TC_BBLK = 512
TC_FBLK = 128


def _make_sc_gather(f0, fs):
    chunk = min(ROWS_PER_WORKER, 32768 // fs)
    nchunk = ROWS_PER_WORKER // chunk

    def body(idx_hbm, table_hbm, out_hbm, idx_v, buf0, buf1, sem0, sem1):
        wid = lax.axis_index("s") * NUM_CORES + lax.axis_index("c")
        base = wid * ROWS_PER_WORKER
        pltpu.sync_copy(idx_hbm.at[pl.ds(base, ROWS_PER_WORKER)], idx_v)

        bufs = (buf0, buf1)
        sems = (sem0, sem1)

        def start_gather(g):
            return pltpu.async_copy(
                table_hbm.at[idx_v.at[pl.ds(g * chunk, chunk)],
                             pl.ds(f0, fs)],
                bufs[g % 2], sems[g % 2])

        pending = start_gather(0)
        for g in range(nchunk):
            nxt = start_gather(g + 1) if g + 1 < nchunk else None
            pending.wait()
            pltpu.sync_copy(bufs[g % 2],
                            out_hbm.at[pl.ds(base + g * chunk, chunk)])
            pending = nxt

    mesh = plsc.VectorSubcoreMesh(core_axis_name="c", subcore_axis_name="s")
    return pl.kernel(
        body,
        out_type=jax.ShapeDtypeStruct((BATCH, fs), jnp.float32),
        mesh=mesh,
        scratch_types=[
            pltpu.VMEM((ROWS_PER_WORKER,), jnp.int32),
            pltpu.VMEM((chunk, fs), jnp.float32),
            pltpu.VMEM((chunk, fs), jnp.float32),
            pltpu.SemaphoreType.DMA,
            pltpu.SemaphoreType.DMA,
        ],
    )


def _tc_transpose_first_body(x_ref, o_ref):
    o_ref[...] = x_ref[...].T


def _tc_transpose_band_body(x_ref, acc_ref, o_ref):
    del acc_ref
    o_ref[...] = x_ref[...].T


def _make_tc_transpose(f0, fs, first):
    grid = (fs // TC_FBLK, BATCH // TC_BBLK)
    in_spec = pl.BlockSpec((TC_BBLK, TC_FBLK), lambda k, i: (i, k))
    out_spec = pl.BlockSpec((TC_FBLK, TC_BBLK),
                            lambda k, i, f=f0 // TC_FBLK: (f + k, i))
    if first:
        return pl.pallas_call(
            _tc_transpose_first_body,
            grid=grid,
            in_specs=[in_spec],
            out_specs=out_spec,
            out_shape=jax.ShapeDtypeStruct((EMB_DIM, BATCH), jnp.float32),
        )
    return pl.pallas_call(
        _tc_transpose_band_body,
        grid=grid,
        in_specs=[in_spec, pl.BlockSpec(memory_space=pltpu.MemorySpace.HBM)],
        out_specs=out_spec,
        out_shape=jax.ShapeDtypeStruct((EMB_DIM, BATCH), jnp.float32),
        input_output_aliases={1: 0},
    )


def kernel(indices, table):
    idx = indices.astype(jnp.int32)
    offsets = [sum(SPLITS[:k]) for k in range(len(SPLITS))]
    parts = [_make_sc_gather(f0, fs)(idx, table)
             for f0, fs in zip(offsets, SPLITS)]
    out_t = _make_tc_transpose(offsets[0], SPLITS[0], True)(parts[0])
    for k in range(1, len(SPLITS)):
        out_t = _make_tc_transpose(offsets[k], SPLITS[k], False)(
            parts[k], out_t)
    return out_t.T.reshape(BATCH, CHANNEL, IM_H, IM_W)


# speedup vs baseline: 2.1842x; 1.1030x over previous
import jax
import jax.numpy as jnp
from jax import lax
from jax.experimental import pallas as pl
from jax.experimental.pallas import tpu as pltpu
from jax.experimental.pallas import tpu_sc as plsc

NUM_ROWS = 100000
EMB_DIM = 3072
BATCH = 8192
CHANNEL, IM_H, IM_W = 3, 32, 32

NUM_CORES = 2
NUM_SUBCORES = 16
NUM_WORKERS = NUM_CORES * NUM_SUBCORES
ROWS_PER_WORKER = BATCH // NUM_WORKERS

SPLITS = (512, 1024, 1024, 512)

TC_BBLK = 512
TC_FBLK = 128


def _make_sc_gather(f0, fs):
    chunk = min(ROWS_PER_WORKER, 32768 // fs)
    nchunk = ROWS_PER_WORKER // chunk

    def body(idx_hbm, table_hbm, out_hbm, idx_v, buf0, buf1, sem0, sem1):
        wid = lax.axis_index("s") * NUM_CORES + lax.axis_index("c")
        base = wid * ROWS_PER_WORKER
        pltpu.sync_copy(idx_hbm.at[pl.ds(base, ROWS_PER_WORKER)], idx_v)

        bufs = (buf0, buf1)
        sems = (sem0, sem1)

        def start_gather(g):
            return pltpu.async_copy(
                table_hbm.at[idx_v.at[pl.ds(g * chunk, chunk)],
                             pl.ds(f0, fs)],
                bufs[g % 2], sems[g % 2])

        pending = start_gather(0)
        for g in range(nchunk):
            nxt = start_gather(g + 1) if g + 1 < nchunk else None
            pending.wait()
            pltpu.sync_copy(bufs[g % 2],
                            out_hbm.at[pl.ds(base + g * chunk, chunk)])
            pending = nxt

    mesh = plsc.VectorSubcoreMesh(core_axis_name="c", subcore_axis_name="s")
    return pl.kernel(
        body,
        out_type=jax.ShapeDtypeStruct((BATCH, fs), jnp.float32),
        mesh=mesh,
        scratch_types=[
            pltpu.VMEM((ROWS_PER_WORKER,), jnp.int32),
            pltpu.VMEM((chunk, fs), jnp.float32),
            pltpu.VMEM((chunk, fs), jnp.float32),
            pltpu.SemaphoreType.DMA,
            pltpu.SemaphoreType.DMA,
        ],
    )


def _tc_transpose_first_body(x_ref, o_ref):
    k = pl.program_id(1)
    o_ref[...] = x_ref[:, pl.ds(k * TC_FBLK, TC_FBLK)].T


def _tc_transpose_band_body(x_ref, acc_ref, o_ref):
    del acc_ref
    k = pl.program_id(1)
    o_ref[...] = x_ref[:, pl.ds(k * TC_FBLK, TC_FBLK)].T


def _make_tc_transpose(f0, fs, first):
    grid = (BATCH // TC_BBLK, fs // TC_FBLK)
    in_spec = pl.BlockSpec((TC_BBLK, fs), lambda i, k: (i, 0))
    out_spec = pl.BlockSpec((TC_FBLK, TC_BBLK),
                            lambda i, k, f=f0 // TC_FBLK: (f + k, i))
    if first:
        return pl.pallas_call(
            _tc_transpose_first_body,
            grid=grid,
            in_specs=[in_spec],
            out_specs=out_spec,
            out_shape=jax.ShapeDtypeStruct((EMB_DIM, BATCH), jnp.float32),
        )
    return pl.pallas_call(
        _tc_transpose_band_body,
        grid=grid,
        in_specs=[in_spec, pl.BlockSpec(memory_space=pltpu.MemorySpace.HBM)],
        out_specs=out_spec,
        out_shape=jax.ShapeDtypeStruct((EMB_DIM, BATCH), jnp.float32),
        input_output_aliases={1: 0},
    )


def kernel(indices, table):
    idx = indices.astype(jnp.int32)
    offsets = [sum(SPLITS[:k]) for k in range(len(SPLITS))]
    parts = [_make_sc_gather(f0, fs)(idx, table)
             for f0, fs in zip(offsets, SPLITS)]
    out_t = _make_tc_transpose(offsets[0], SPLITS[0], True)(parts[0])
    for k in range(1, len(SPLITS)):
        out_t = _make_tc_transpose(offsets[k], SPLITS[k], False)(
            parts[k], out_t)
    return out_t.T.reshape(BATCH, CHANNEL, IM_H, IM_W)


# speedup vs baseline: 3.9773x; 1.8209x over previous
import jax
import jax.numpy as jnp
from jax import lax
from jax.experimental import pallas as pl
from jax.experimental.pallas import tpu as pltpu
from jax.experimental.pallas import tpu_sc as plsc

NUM_ROWS = 100000
EMB_DIM = 3072
BATCH = 8192
CHANNEL, IM_H, IM_W = 3, 32, 32

NUM_CORES = 2
NUM_SUBCORES = 16
NUM_WORKERS = NUM_CORES * NUM_SUBCORES
NSPLIT = 2
FSLICE = EMB_DIM // NSPLIT
ROWS_PER_WORKER = BATCH // NUM_WORKERS
CHUNK = max(f for f in (8, 16, 32, 64) if f * FSLICE <= 32768)
NCHUNK = ROWS_PER_WORKER // CHUNK

TC_BBLK = 512


def _make_sc_gather(split):
    f0 = split * FSLICE

    def body(idx_hbm, table_hbm, out_hbm, idx_v, buf0, buf1, sem0, sem1):
        wid = lax.axis_index("s") * NUM_CORES + lax.axis_index("c")
        base = wid * ROWS_PER_WORKER
        pltpu.sync_copy(idx_hbm.at[pl.ds(base, ROWS_PER_WORKER)], idx_v)

        bufs = (buf0, buf1)
        sems = (sem0, sem1)

        def start_gather(g):
            return pltpu.async_copy(
                table_hbm.at[idx_v.at[pl.ds(g * CHUNK, CHUNK)],
                             pl.ds(f0, FSLICE)],
                bufs[g % 2], sems[g % 2])

        pending = start_gather(0)
        for g in range(NCHUNK):
            nxt = start_gather(g + 1) if g + 1 < NCHUNK else None
            pending.wait()
            pltpu.sync_copy(bufs[g % 2],
                            out_hbm.at[pl.ds(base + g * CHUNK, CHUNK)])
            pending = nxt

    mesh = plsc.VectorSubcoreMesh(core_axis_name="c", subcore_axis_name="s")
    return pl.kernel(
        body,
        out_type=jax.ShapeDtypeStruct((BATCH, FSLICE), jnp.float32),
        mesh=mesh,
        scratch_types=[
            pltpu.VMEM((ROWS_PER_WORKER,), jnp.int32),
            pltpu.VMEM((CHUNK, FSLICE), jnp.float32),
            pltpu.VMEM((CHUNK, FSLICE), jnp.float32),
            pltpu.SemaphoreType.DMA,
            pltpu.SemaphoreType.DMA,
        ],
    )


def _tc_transpose_first_body(x_ref, o_ref):
    o_ref[...] = x_ref[...].T


def _tc_transpose_band_body(x_ref, acc_ref, o_ref):
    del acc_ref
    o_ref[...] = x_ref[...].T


def _make_tc_transpose(split):
    if split == 0:
        return pl.pallas_call(
            _tc_transpose_first_body,
            grid=(BATCH // TC_BBLK,),
            in_specs=[pl.BlockSpec((TC_BBLK, FSLICE), lambda i: (i, 0))],
            out_specs=pl.BlockSpec((FSLICE, TC_BBLK), lambda i: (0, i)),
            out_shape=jax.ShapeDtypeStruct((EMB_DIM, BATCH), jnp.float32),
        )
    return pl.pallas_call(
        _tc_transpose_band_body,
        grid=(BATCH // TC_BBLK,),
        in_specs=[
            pl.BlockSpec((TC_BBLK, FSLICE), lambda i: (i, 0)),
            pl.BlockSpec(memory_space=pltpu.MemorySpace.HBM),
        ],
        out_specs=pl.BlockSpec((FSLICE, TC_BBLK), lambda i, s=split: (s, i)),
        out_shape=jax.ShapeDtypeStruct((EMB_DIM, BATCH), jnp.float32),
        input_output_aliases={1: 0},
    )


def kernel(indices, table):
    idx = indices.astype(jnp.int32)
    parts = [_make_sc_gather(k)(idx, table) for k in range(NSPLIT)]
    out_t = _make_tc_transpose(0)(parts[0])
    for k in range(1, NSPLIT):
        out_t = _make_tc_transpose(k)(parts[k], out_t)
    return out_t.T.reshape(BATCH, CHANNEL, IM_H, IM_W)


# speedup vs baseline: 4.0261x; 1.0123x over previous
import jax
import jax.numpy as jnp
from jax import lax
from jax.experimental import pallas as pl
from jax.experimental.pallas import tpu as pltpu
from jax.experimental.pallas import tpu_sc as plsc

NUM_ROWS = 100000
EMB_DIM = 3072
BATCH = 8192
CHANNEL, IM_H, IM_W = 3, 32, 32

NUM_CORES = 2
NUM_SUBCORES = 16
NUM_WORKERS = NUM_CORES * NUM_SUBCORES
NSPLIT = 2
FSLICE = EMB_DIM // NSPLIT
ROWS_PER_WORKER = BATCH // NUM_WORKERS
CHUNK = max(f for f in (8, 16, 32, 64) if f * FSLICE <= 32768)
NCHUNK = ROWS_PER_WORKER // CHUNK

TC_BBLK = 1024


def _make_sc_gather(split):
    f0 = split * FSLICE

    def body(idx_hbm, table_hbm, out_hbm, idx_v, buf0, buf1, sem0, sem1):
        wid = lax.axis_index("s") * NUM_CORES + lax.axis_index("c")
        base = wid * ROWS_PER_WORKER
        pltpu.sync_copy(idx_hbm.at[pl.ds(base, ROWS_PER_WORKER)], idx_v)

        bufs = (buf0, buf1)
        sems = (sem0, sem1)

        def start_gather(g):
            return pltpu.async_copy(
                table_hbm.at[idx_v.at[pl.ds(g * CHUNK, CHUNK)],
                             pl.ds(f0, FSLICE)],
                bufs[g % 2], sems[g % 2])

        pending = start_gather(0)
        for g in range(NCHUNK):
            nxt = start_gather(g + 1) if g + 1 < NCHUNK else None
            pending.wait()
            pltpu.sync_copy(bufs[g % 2],
                            out_hbm.at[pl.ds(base + g * CHUNK, CHUNK)])
            pending = nxt

    mesh = plsc.VectorSubcoreMesh(core_axis_name="c", subcore_axis_name="s")
    return pl.kernel(
        body,
        out_type=jax.ShapeDtypeStruct((BATCH, FSLICE), jnp.float32),
        mesh=mesh,
        scratch_types=[
            pltpu.VMEM((ROWS_PER_WORKER,), jnp.int32),
            pltpu.VMEM((CHUNK, FSLICE), jnp.float32),
            pltpu.VMEM((CHUNK, FSLICE), jnp.float32),
            pltpu.SemaphoreType.DMA,
            pltpu.SemaphoreType.DMA,
        ],
    )


def _tc_transpose_first_body(x_ref, o_ref):
    o_ref[...] = x_ref[...].T


def _tc_transpose_band_body(x_ref, acc_ref, o_ref):
    del acc_ref
    o_ref[...] = x_ref[...].T


def _make_tc_transpose(split):
    if split == 0:
        return pl.pallas_call(
            _tc_transpose_first_body,
            grid=(BATCH // TC_BBLK,),
            in_specs=[pl.BlockSpec((TC_BBLK, FSLICE), lambda i: (i, 0))],
            out_specs=pl.BlockSpec((FSLICE, TC_BBLK), lambda i: (0, i)),
            out_shape=jax.ShapeDtypeStruct((EMB_DIM, BATCH), jnp.float32),
        )
    return pl.pallas_call(
        _tc_transpose_band_body,
        grid=(BATCH // TC_BBLK,),
        in_specs=[
            pl.BlockSpec((TC_BBLK, FSLICE), lambda i: (i, 0)),
            pl.BlockSpec(memory_space=pltpu.MemorySpace.HBM),
        ],
        out_specs=pl.BlockSpec((FSLICE, TC_BBLK), lambda i, s=split: (s, i)),
        out_shape=jax.ShapeDtypeStruct((EMB_DIM, BATCH), jnp.float32),
        input_output_aliases={1: 0},
    )


def kernel(indices, table):
    idx = indices.astype(jnp.int32)
    parts = [_make_sc_gather(k)(idx, table) for k in range(NSPLIT)]
    out_t = _make_tc_transpose(0)(parts[0])
    for k in range(1, NSPLIT):
        out_t = _make_tc_transpose(k)(parts[k], out_t)
    return out_t.T.reshape(BATCH, CHANNEL, IM_H, IM_W)


# speedup vs baseline: 4.0590x; 1.0082x over previous
import jax
import jax.numpy as jnp
from jax import lax
from jax.experimental import pallas as pl
from jax.experimental.pallas import tpu as pltpu
from jax.experimental.pallas import tpu_sc as plsc

NUM_ROWS = 100000
EMB_DIM = 3072
BATCH = 8192
CHANNEL, IM_H, IM_W = 3, 32, 32

NUM_CORES = 2
NUM_SUBCORES = 16
NUM_WORKERS = NUM_CORES * NUM_SUBCORES
NSPLIT = 2
FSLICE = EMB_DIM // NSPLIT
ROWS_PER_WORKER = BATCH // NUM_WORKERS
CHUNK = max(f for f in (8, 16, 32, 64) if f * FSLICE <= 32768)
NCHUNK = ROWS_PER_WORKER // CHUNK

TC_BBLK = 2048


def _make_sc_gather(split):
    f0 = split * FSLICE

    def body(idx_hbm, table_hbm, out_hbm, idx_v, buf0, buf1, sem0, sem1):
        wid = lax.axis_index("s") * NUM_CORES + lax.axis_index("c")
        base = wid * ROWS_PER_WORKER
        pltpu.sync_copy(idx_hbm.at[pl.ds(base, ROWS_PER_WORKER)], idx_v)

        bufs = (buf0, buf1)
        sems = (sem0, sem1)

        def start_gather(g):
            return pltpu.async_copy(
                table_hbm.at[idx_v.at[pl.ds(g * CHUNK, CHUNK)],
                             pl.ds(f0, FSLICE)],
                bufs[g % 2], sems[g % 2])

        pending = start_gather(0)
        for g in range(NCHUNK):
            nxt = start_gather(g + 1) if g + 1 < NCHUNK else None
            pending.wait()
            pltpu.sync_copy(bufs[g % 2],
                            out_hbm.at[pl.ds(base + g * CHUNK, CHUNK)])
            pending = nxt

    mesh = plsc.VectorSubcoreMesh(core_axis_name="c", subcore_axis_name="s")
    return pl.kernel(
        body,
        out_type=jax.ShapeDtypeStruct((BATCH, FSLICE), jnp.float32),
        mesh=mesh,
        scratch_types=[
            pltpu.VMEM((ROWS_PER_WORKER,), jnp.int32),
            pltpu.VMEM((CHUNK, FSLICE), jnp.float32),
            pltpu.VMEM((CHUNK, FSLICE), jnp.float32),
            pltpu.SemaphoreType.DMA,
            pltpu.SemaphoreType.DMA,
        ],
    )


def _tc_transpose_first_body(x_ref, o_ref):
    o_ref[...] = x_ref[...].T


def _tc_transpose_band_body(x_ref, acc_ref, o_ref):
    del acc_ref
    o_ref[...] = x_ref[...].T


def _make_tc_transpose(split):
    if split == 0:
        return pl.pallas_call(
            _tc_transpose_first_body,
            grid=(BATCH // TC_BBLK,),
            in_specs=[pl.BlockSpec((TC_BBLK, FSLICE), lambda i: (i, 0))],
            out_specs=pl.BlockSpec((FSLICE, TC_BBLK), lambda i: (0, i)),
            out_shape=jax.ShapeDtypeStruct((EMB_DIM, BATCH), jnp.float32),
        )
    return pl.pallas_call(
        _tc_transpose_band_body,
        grid=(BATCH // TC_BBLK,),
        in_specs=[
            pl.BlockSpec((TC_BBLK, FSLICE), lambda i: (i, 0)),
            pl.BlockSpec(memory_space=pltpu.MemorySpace.HBM),
        ],
        out_specs=pl.BlockSpec((FSLICE, TC_BBLK), lambda i, s=split: (s, i)),
        out_shape=jax.ShapeDtypeStruct((EMB_DIM, BATCH), jnp.float32),
        input_output_aliases={1: 0},
    )


def kernel(indices, table):
    idx = indices.astype(jnp.int32)
    parts = [_make_sc_gather(k)(idx, table) for k in range(NSPLIT)]
    out_t = _make_tc_transpose(0)(parts[0])
    for k in range(1, NSPLIT):
        out_t = _make_tc_transpose(k)(parts[k], out_t)
    return out_t.T.reshape(BATCH, CHANNEL, IM_H, IM_W)
